# baseline (speedup 1.0000x reference)
import functools

import jax
import jax.numpy as jnp
from jax import lax
from jax.experimental import pallas as pl
from jax.experimental.pallas import tpu as pltpu
from jax.experimental.pallas import tpu_sc as plsc

N = 10000
K = 16
E = 160000
DN = 128
DE = 16
H = 128
NK = N * K

NC = 2
NS = 16
NW = NC * NS

_MESH = dict(core_axis_name="c", subcore_axis_name="s")
_SC_PARAMS = pltpu.CompilerParams(needs_layout_passes=False)

IDS_PER_W = E // NW
LP_CHUNK = 16000
LP_NCH = NK // LP_CHUNK


def _wid():
    return lax.axis_index("s") * NC + lax.axis_index("c")


@functools.partial(
    pl.kernel,
    out_type=jax.ShapeDtypeStruct((E,), jnp.int32),
    mesh=plsc.VectorSubcoreMesh(**_MESH),
    compiler_params=_SC_PARAMS,
    scratch_types=[
        pltpu.VMEM((IDS_PER_W,), jnp.int32),
        pltpu.VMEM((LP_CHUNK,), jnp.int32),
        pltpu.VMEM((LP_CHUNK,), jnp.int32),
        pltpu.SemaphoreType.DMA,
        pltpu.SemaphoreType.DMA,
    ],
)
def _lastpos_sc(en_hbm, out_hbm, lp_v, ids0, ids1, sem0, sem1):
    wid = _wid()
    lo = wid * IDS_PER_W
    iota = lax.iota(jnp.int32, 16)
    ids = (ids0, ids1)
    sem = (sem0, sem1)

    def issue(c, b):
        @pl.when(c < LP_NCH)
        def _():
            pltpu.async_copy(en_hbm.at[pl.ds(c * LP_CHUNK, LP_CHUNK)],
                             ids[b], sem[b])

    def wait(c, b):
        pltpu.make_async_copy(en_hbm.at[pl.ds(c * LP_CHUNK, LP_CHUNK)],
                              ids[b], sem[b]).wait()

    def compute(c, b):
        ids_v = ids[b]

        def vec_body(i, carry2):
            for u in range(8):
                off = i * 128 + u * 16
                v = ids_v[pl.ds(off, 16)]
                local = v - lo
                m = local.astype(jnp.uint32) < jnp.uint32(IDS_PER_W)
                _, last = plsc.scan_count(v, mask=m)
                pos = (c * LP_CHUNK + off) + iota
                plsc.store_scatter(lp_v, [local], pos, mask=last)
            return carry2

        lax.fori_loop(0, LP_CHUNK // 128, vec_body, 0)

    issue(0, 0)

    def pair_body(cc, carry):
        for b in range(2):
            c = cc * 2 + b
            wait(c, b)
            issue(c + 1, 1 - b)
            compute(c, b)
        return carry

    lax.fori_loop(0, LP_NCH // 2, pair_body, 0)
    pltpu.sync_copy(lp_v, out_hbm.at[pl.ds(lo, IDS_PER_W)])


A_CN = 8
A_CP = A_CN * K
A_NCH = N // A_CN
A_MAIN = A_NCH // NW
A_TAIL = A_NCH - A_MAIN * NW
A_SLAB = (A_MAIN + 1) * A_CP


@functools.partial(
    pl.kernel,
    out_type=(
        jax.ShapeDtypeStruct((N, DN), jnp.float32),
        jax.ShapeDtypeStruct((N, DN), jnp.float32),
    ),
    mesh=plsc.VectorSubcoreMesh(**_MESH),
    compiler_params=_SC_PARAMS,
    scratch_types=[
        pltpu.VMEM((A_SLAB,), jnp.int32),
        pltpu.VMEM((A_SLAB,), jnp.int32),
        pltpu.VMEM((A_CP, DN), jnp.float32),
        pltpu.VMEM((A_CP, DN), jnp.float32),
        pltpu.VMEM((A_CP, DN), jnp.float32),
        pltpu.VMEM((A_CP, DN), jnp.float32),
        pltpu.VMEM((A_CN, DN), jnp.float32),
        pltpu.VMEM((A_CN, DN), jnp.float32),
        pltpu.SemaphoreType.DMA,
        pltpu.SemaphoreType.DMA,
        pltpu.SemaphoreType.DMA,
        pltpu.SemaphoreType.DMA,
    ],
)
def _agg1_sc(nn_hbm, en_hbm, x_hbm, e_hbm, mnx_hbm, mne_hbm,
             nnL, enL, xr0, xr1, er0, er1, ox_v, oe_v,
             semx0, semx1, seme0, seme1):
    wid = _wid()
    scale = jnp.float32(1.0 / K)
    xr = (xr0, xr1)
    er = (er0, er1)
    semx = (semx0, semx1)
    seme = (seme0, seme1)

    base_pos = wid * (A_MAIN * A_CP)
    pltpu.sync_copy(nn_hbm.at[pl.ds(base_pos, A_MAIN * A_CP)],
                    nnL.at[pl.ds(0, A_MAIN * A_CP)])
    pltpu.sync_copy(en_hbm.at[pl.ds(base_pos, A_MAIN * A_CP)],
                    enL.at[pl.ds(0, A_MAIN * A_CP)])

    @pl.when(wid < A_TAIL)
    def _():
        tail_pos = (A_MAIN * NW + wid) * A_CP
        pltpu.sync_copy(nn_hbm.at[pl.ds(tail_pos, A_CP)],
                        nnL.at[pl.ds(A_MAIN * A_CP, A_CP)])
        pltpu.sync_copy(en_hbm.at[pl.ds(tail_pos, A_CP)],
                        enL.at[pl.ds(A_MAIN * A_CP, A_CP)])

    def valid(t):
        return (t < A_MAIN) | ((t == A_MAIN) & (wid < A_TAIL))

    def cid_of(t):
        return jnp.where(t == A_MAIN, A_MAIN * NW + wid, wid * A_MAIN + t)

    def issue(t, b):
        @pl.when(valid(t))
        def _():
            sl = pl.ds(t * A_CP, A_CP)
            pltpu.async_copy(x_hbm.at[nnL.at[sl]], xr[b], semx[b])
            pltpu.async_copy(e_hbm.at[enL.at[sl]], er[b], seme[b])

    def wait(t, b):
        @pl.when(valid(t))
        def _():
            sl = pl.ds(t * A_CP, A_CP)
            pltpu.make_async_copy(x_hbm.at[nnL.at[sl]], xr[b], semx[b]).wait()
            pltpu.make_async_copy(e_hbm.at[enL.at[sl]], er[b], seme[b]).wait()

    def compute(t, b):
        @pl.when(valid(t))
        def _():
            cid = cid_of(t)

            def node_body(i, carry2):
                base = i * K
                for cg in range(DN // 16):
                    sl = pl.ds(cg * 16, 16)
                    acc = xr[b][base, sl]
                    acce = er[b][base, sl]
                    for k in range(1, K):
                        acc = acc + xr[b][base + k, sl]
                        acce = acce + er[b][base + k, sl]
                    ox_v[i, sl] = acc * scale
                    oe_v[i, sl] = acce * scale
                return carry2

            lax.fori_loop(0, A_CN, node_body, 0)
            row_off = pl.multiple_of(cid * A_CN, 8)
            pltpu.sync_copy(ox_v, mnx_hbm.at[pl.ds(row_off, A_CN)])
            pltpu.sync_copy(oe_v, mne_hbm.at[pl.ds(row_off, A_CN)])

    issue(0, 0)

    def pair_body(tt, carry):
        for b in range(2):
            t = tt * 2 + b
            wait(t, b)
            issue(t + 1, 1 - b)
            compute(t, b)
        return carry

    lax.fori_loop(0, (A_MAIN + 1) // 2, pair_body, 0)


B_CN = 5
B_CP = B_CN * K
B_NCH = N // B_CN
B_MAIN = B_NCH // NW
B_TAIL = B_NCH - B_MAIN * NW
B_SLAB = (B_MAIN + 1) * B_CP


@functools.partial(
    pl.kernel,
    out_type=(
        jax.ShapeDtypeStruct((N * H,), jnp.float32),
        jax.ShapeDtypeStruct((N * DN,), jnp.float32),
    ),
    mesh=plsc.VectorSubcoreMesh(**_MESH),
    compiler_params=_SC_PARAMS,
    scratch_types=[
        pltpu.VMEM((B_SLAB,), jnp.int32),
        pltpu.VMEM((B_SLAB,), jnp.int32),
        pltpu.VMEM((B_SLAB,), jnp.int32),
        pltpu.VMEM((B_SLAB,), jnp.int32),
        pltpu.VMEM((B_SLAB,), jnp.int32),
        pltpu.VMEM((B_CP, H), jnp.float32),
        pltpu.VMEM((B_CP, H), jnp.float32),
        pltpu.VMEM((B_CP, H), jnp.float32),
        pltpu.VMEM((B_CP, H), jnp.float32),
        pltpu.VMEM((B_CP, H), jnp.float32),
        pltpu.VMEM((B_CP, H), jnp.float32),
        pltpu.VMEM((B_CP, DN), jnp.float32),
        pltpu.VMEM((B_CP, DN), jnp.float32),
        pltpu.VMEM((B_CN * H,), jnp.float32),
        pltpu.VMEM((B_CN * DN,), jnp.float32),
        pltpu.SemaphoreType.DMA,
        pltpu.SemaphoreType.DMA,
        pltpu.SemaphoreType.DMA,
        pltpu.SemaphoreType.DMA,
        pltpu.SemaphoreType.DMA,
        pltpu.SemaphoreType.DMA,
        pltpu.SemaphoreType.DMA,
        pltpu.SemaphoreType.DMA,
        pltpu.SemaphoreType.DMA,
        pltpu.SemaphoreType.DMA,
    ],
)
def _edge2_sc(nn_hbm, en_hbm, lp_hbm, a_hbm, b_hbm, c_hbm, x1_hbm,
              me2_hbm, mnx2_hbm,
              enL, nnL, gL, aiL, biL,
              ar0, ar1, br0, br1, cr0, cr1, xr0, xr1,
              ome_v, omx_v,
              semg, sembi, sa0, sa1, sb0, sb1, sc0, sc1, sx0, sx1):
    wid = _wid()
    scale = jnp.float32(1.0 / K)
    ar = (ar0, ar1)
    br = (br0, br1)
    cr = (cr0, cr1)
    xr = (xr0, xr1)
    sa = (sa0, sa1)
    sb = (sb0, sb1)
    sc = (sc0, sc1)
    sx = (sx0, sx1)

    main_pos = B_MAIN * B_CP
    base_pos = wid * main_pos
    pltpu.sync_copy(en_hbm.at[pl.ds(base_pos, main_pos)],
                    enL.at[pl.ds(0, main_pos)])
    pltpu.sync_copy(nn_hbm.at[pl.ds(base_pos, main_pos)],
                    nnL.at[pl.ds(0, main_pos)])

    @pl.when(wid < B_TAIL)
    def _():
        tail_pos = (B_MAIN * NW + wid) * B_CP
        pltpu.sync_copy(en_hbm.at[pl.ds(tail_pos, B_CP)],
                        enL.at[pl.ds(main_pos, B_CP)])
        pltpu.sync_copy(nn_hbm.at[pl.ds(tail_pos, B_CP)],
                        nnL.at[pl.ds(main_pos, B_CP)])

    def valid(t):
        return (t < B_MAIN) | ((t == B_MAIN) & (wid < B_TAIL))

    def cid_of(t):
        return jnp.where(t == B_MAIN, B_MAIN * NW + wid, wid * B_MAIN + t)

    n_trips = B_MAIN + 1

    def ga_issue(t, carry):
        @pl.when(valid(t))
        def _():
            sl = pl.ds(t * B_CP, B_CP)
            pltpu.async_copy(lp_hbm.at[enL.at[sl]], gL.at[sl], semg)
        return carry

    lax.fori_loop(0, n_trips, ga_issue, 0)

    def ga_drain(t, carry):
        @pl.when(valid(t))
        def _():
            sl = pl.ds(t * B_CP, B_CP)
            pltpu.make_async_copy(lp_hbm.at[enL.at[sl]], gL.at[sl],
                                  semg).wait()

            def sh_body(i, carry2):
                off = t * B_CP + i * 16
                aiL[pl.ds(off, 16)] = lax.shift_right_logical(
                    gL[pl.ds(off, 16)], 4)
                return carry2

            lax.fori_loop(0, B_CP // 16, sh_body, 0)
            pltpu.async_copy(nn_hbm.at[gL.at[sl]], biL.at[sl], sembi)
        return carry

    lax.fori_loop(0, n_trips, ga_drain, 0)

    def bi_drain(t, carry):
        @pl.when(valid(t))
        def _():
            sl = pl.ds(t * B_CP, B_CP)
            pltpu.make_async_copy(nn_hbm.at[gL.at[sl]], biL.at[sl],
                                  sembi).wait()
        return carry

    lax.fori_loop(0, n_trips, bi_drain, 0)

    def issue(t, b):
        @pl.when(valid(t))
        def _():
            sl = pl.ds(t * B_CP, B_CP)
            pltpu.async_copy(a_hbm.at[aiL.at[sl]], ar[b], sa[b])
            pltpu.async_copy(b_hbm.at[biL.at[sl]], br[b], sb[b])
            pltpu.async_copy(c_hbm.at[enL.at[sl]], cr[b], sc[b])
            pltpu.async_copy(x1_hbm.at[nnL.at[sl]], xr[b], sx[b])

    def wait(t, b):
        @pl.when(valid(t))
        def _():
            sl = pl.ds(t * B_CP, B_CP)
            pltpu.make_async_copy(a_hbm.at[aiL.at[sl]], ar[b], sa[b]).wait()
            pltpu.make_async_copy(b_hbm.at[biL.at[sl]], br[b], sb[b]).wait()
            pltpu.make_async_copy(c_hbm.at[enL.at[sl]], cr[b], sc[b]).wait()
            pltpu.make_async_copy(x1_hbm.at[nnL.at[sl]], xr[b], sx[b]).wait()

    def compute(t, b):
        @pl.when(valid(t))
        def _():
            cid = cid_of(t)

            def node_body(i, carry2):
                base = i * K
                for cg in range(H // 16):
                    sl = pl.ds(cg * 16, 16)
                    acce = jnp.maximum(
                        ar[b][base, sl] + br[b][base, sl] + cr[b][base, sl],
                        0.0)
                    accx = xr[b][base, sl]
                    for k in range(1, K):
                        acce = acce + jnp.maximum(
                            ar[b][base + k, sl] + br[b][base + k, sl]
                            + cr[b][base + k, sl], 0.0)
                        accx = accx + xr[b][base + k, sl]
                    ome_v[pl.ds(i * H + cg * 16, 16)] = acce * scale
                    omx_v[pl.ds(i * DN + cg * 16, 16)] = accx * scale
                return carry2

            lax.fori_loop(0, B_CN, node_body, 0)
            out_off = pl.multiple_of(cid * (B_CN * H), 128)
            pltpu.sync_copy(ome_v, me2_hbm.at[pl.ds(out_off, B_CN * H)])
            pltpu.sync_copy(omx_v, mnx2_hbm.at[pl.ds(out_off, B_CN * DN)])

    issue(0, 0)

    def pair_body(tt, carry):
        for b in range(2):
            t = tt * 2 + b
            wait(t, b)
            issue(t + 1, 1 - b)
            compute(t, b)
        return carry

    lax.fori_loop(0, (n_trips + 1) // 2, pair_body, 0)


_BN = 400
_BE = 2000


def _dot(a, b):
    return jax.lax.dot_general(a, b, (((1,), (0,)), ((), ())),
                               preferred_element_type=jnp.float32)


def _tc_ab_body(x_ref, wea_ref, web_ref, a_ref, b_ref):
    x = x_ref[...]
    a_ref[...] = _dot(x, wea_ref[...])
    b_ref[...] = _dot(x, web_ref[...])


def _tc_ab(x0, wea, web):
    row = lambda i: (i, 0)
    full = lambda i: (0, 0)
    return pl.pallas_call(
        _tc_ab_body,
        grid=(N // _BN,),
        in_specs=[
            pl.BlockSpec((_BN, DN), row),
            pl.BlockSpec((DN, H), full),
            pl.BlockSpec((DN, H), full),
        ],
        out_specs=[
            pl.BlockSpec((_BN, H), row),
            pl.BlockSpec((_BN, H), row),
        ],
        out_shape=[
            jax.ShapeDtypeStruct((N, H), jnp.float32),
            jax.ShapeDtypeStruct((N, H), jnp.float32),
        ],
    )(x0, wea, web)


def _tc_node1_body(x_ref, mnx_ref, mnd_ref, wna_ref, wnb_ref, bn_ref,
                   w2a_ref, x1_ref, p_ref):
    acc = _dot(x_ref[...], wna_ref[...])
    acc = acc + _dot(mnx_ref[...], wnb_ref[...])
    x1 = jnp.maximum(acc + mnd_ref[...] + bn_ref[...], 0.0)
    x1_ref[...] = x1
    p_ref[...] = _dot(x1, w2a_ref[...])


def _tc_node1(x0, mnx1, mnd1, wna, wnb, bn1, w2a):
    row = lambda i: (i, 0)
    full = lambda i: (0, 0)
    return pl.pallas_call(
        _tc_node1_body,
        grid=(N // _BN,),
        in_specs=[
            pl.BlockSpec((_BN, DN), row),
            pl.BlockSpec((_BN, DN), row),
            pl.BlockSpec((_BN, H), row),
            pl.BlockSpec((DN, H), full),
            pl.BlockSpec((DN, H), full),
            pl.BlockSpec((1, H), full),
            pl.BlockSpec((H, H), full),
        ],
        out_specs=[
            pl.BlockSpec((_BN, H), row),
            pl.BlockSpec((_BN, H), row),
        ],
        out_shape=[
            jax.ShapeDtypeStruct((N, H), jnp.float32),
            jax.ShapeDtypeStruct((N, H), jnp.float32),
        ],
    )(x0, mnx1, mnd1, wna, wnb, bn1, w2a)


def _tc_edge1_body(e_ref, wec_ref, be_ref, wnc_ref, c_ref, d_ref):
    e = e_ref[...]
    c_ref[...] = _dot(e, wec_ref[...]) + be_ref[...]
    d_ref[...] = _dot(e, wnc_ref[...])


def _tc_edge1(e0, wec, be1, wnc):
    return pl.pallas_call(
        _tc_edge1_body,
        grid=(E // _BE,),
        in_specs=[
            pl.BlockSpec((_BE, DE), lambda i: (i, 0)),
            pl.BlockSpec((DE, H), lambda i: (0, 0)),
            pl.BlockSpec((1, H), lambda i: (0, 0)),
            pl.BlockSpec((DE, H), lambda i: (0, 0)),
        ],
        out_specs=[
            pl.BlockSpec((_BE, H), lambda i: (i, 0)),
            pl.BlockSpec((_BE, H), lambda i: (i, 0)),
        ],
        out_shape=[
            jax.ShapeDtypeStruct((E, H), jnp.float32),
            jax.ShapeDtypeStruct((E, H), jnp.float32),
        ],
    )(e0, wec, be1, wnc)


def _tc_node2_body(p_ref, mnx_ref, me_ref, wb_ref, wc_ref, bn_ref,
                   wf_ref, bf_ref, out_ref):
    acc = p_ref[...]
    acc = acc + _dot(mnx_ref[...], wb_ref[...])
    acc = acc + _dot(me_ref[...], wc_ref[...])
    x2 = jnp.maximum(acc + bn_ref[...], 0.0)
    out_ref[...] = _dot(x2, wf_ref[...]) + bf_ref[...]


def _tc_node2(p1, mnx2, me2, wb, wc, bn2, wf, bf):
    row = lambda i: (i, 0)
    full = lambda i: (0, 0)
    return pl.pallas_call(
        _tc_node2_body,
        grid=(N // _BN,),
        in_specs=[
            pl.BlockSpec((_BN, H), row),
            pl.BlockSpec((_BN, DN), row),
            pl.BlockSpec((_BN, H), row),
            pl.BlockSpec((DN, H), full),
            pl.BlockSpec((H, H), full),
            pl.BlockSpec((1, H), full),
            pl.BlockSpec((H, H), full),
            pl.BlockSpec((1, H), full),
        ],
        out_specs=pl.BlockSpec((_BN, H), row),
        out_shape=jax.ShapeDtypeStruct((N, H), jnp.float32),
    )(p1, mnx2, me2, wb, wc, bn2, wf, bf)


def kernel(node_neighbors, edge_neighbors, node_feats, edge_feats,
           Wn1, bn1, We1, be1, Wn2, bn2, We2, be2, Wf, bf):
    nn_f = node_neighbors.reshape(-1).astype(jnp.int32)
    en_f = edge_neighbors.reshape(-1).astype(jnp.int32)
    x0 = node_feats
    e0 = edge_feats

    lastpos = _lastpos_sc(en_f)
    C1, D1 = _tc_edge1(e0, We1[2 * DN:], be1.reshape(1, H), Wn1[2 * DN:])
    A1, B1 = _tc_ab(x0, We1[:DN], We1[DN:2 * DN])

    mnx1, mnd1 = _agg1_sc(nn_f, en_f, x0, D1)

    x1, P1 = _tc_node1(x0, mnx1, mnd1, Wn1[:DN], Wn1[DN:2 * DN],
                       bn1.reshape(1, H), Wn2[:H])

    me2_f, mnx2_f = _edge2_sc(nn_f, en_f, lastpos, A1, B1, C1, x1)
    me2 = me2_f.reshape(N, H)
    mnx2 = mnx2_f.reshape(N, DN)

    out = _tc_node2(
        P1, mnx2, me2,
        Wn2[H:2 * H], Wn2[2 * H:],
        bn2.reshape(1, H), Wf, bf.reshape(1, H))
    return out


# speedup vs baseline: 1.2316x; 1.2316x over previous
import functools

import jax
import jax.numpy as jnp
from jax import lax
from jax.experimental import pallas as pl
from jax.experimental.pallas import tpu as pltpu
from jax.experimental.pallas import tpu_sc as plsc

N = 10000
K = 16
E = 160000
DN = 128
DE = 16
H = 128
NK = N * K

NC = 2
NS = 16
NW = NC * NS

_MESH = dict(core_axis_name="c", subcore_axis_name="s")
_SC_PARAMS = pltpu.CompilerParams(needs_layout_passes=False)

IDS_PER_W = E // NW
LP_CHUNK = 16000
LP_NCH = NK // LP_CHUNK


def _wid():
    return lax.axis_index("s") * NC + lax.axis_index("c")


@functools.partial(
    pl.kernel,
    out_type=jax.ShapeDtypeStruct((E,), jnp.int32),
    mesh=plsc.VectorSubcoreMesh(**_MESH),
    compiler_params=_SC_PARAMS,
    scratch_types=[
        pltpu.VMEM((IDS_PER_W,), jnp.int32),
        pltpu.VMEM((LP_CHUNK,), jnp.int32),
        pltpu.VMEM((LP_CHUNK,), jnp.int32),
        pltpu.SemaphoreType.DMA,
        pltpu.SemaphoreType.DMA,
    ],
)
def _lastpos_sc(en_hbm, out_hbm, lp_v, ids0, ids1, sem0, sem1):
    wid = _wid()
    lo = wid * IDS_PER_W
    iota = lax.iota(jnp.int32, 16)
    ids = (ids0, ids1)
    sem = (sem0, sem1)

    def issue(c, b):
        @pl.when(c < LP_NCH)
        def _():
            pltpu.async_copy(en_hbm.at[pl.ds(c * LP_CHUNK, LP_CHUNK)],
                             ids[b], sem[b])

    def wait(c, b):
        pltpu.make_async_copy(en_hbm.at[pl.ds(c * LP_CHUNK, LP_CHUNK)],
                              ids[b], sem[b]).wait()

    def compute(c, b):
        ids_v = ids[b]

        def vec_body(i, carry2):
            for u in range(8):
                off = i * 128 + u * 16
                v = ids_v[pl.ds(off, 16)]
                local = v - lo
                m = local.astype(jnp.uint32) < jnp.uint32(IDS_PER_W)
                _, last = plsc.scan_count(v, mask=m)
                pos = (c * LP_CHUNK + off) + iota
                plsc.store_scatter(lp_v, [local], pos, mask=last)
            return carry2

        lax.fori_loop(0, LP_CHUNK // 128, vec_body, 0)

    issue(0, 0)

    def pair_body(cc, carry):
        for b in range(2):
            c = cc * 2 + b
            wait(c, b)
            issue(c + 1, 1 - b)
            compute(c, b)
        return carry

    lax.fori_loop(0, LP_NCH // 2, pair_body, 0)
    pltpu.sync_copy(lp_v, out_hbm.at[pl.ds(lo, IDS_PER_W)])


A_CN = 8
A_CP = A_CN * K
A_NCH = N // A_CN
A_MAIN = A_NCH // NW
A_TAIL = A_NCH - A_MAIN * NW
A_SLAB = (A_MAIN + 1) * A_CP


@functools.partial(
    pl.kernel,
    out_type=(
        jax.ShapeDtypeStruct((N, DN), jnp.float32),
        jax.ShapeDtypeStruct((N, DN), jnp.float32),
    ),
    mesh=plsc.VectorSubcoreMesh(**_MESH),
    compiler_params=_SC_PARAMS,
    scratch_types=[
        pltpu.VMEM((A_SLAB,), jnp.int32),
        pltpu.VMEM((A_SLAB,), jnp.int32),
        pltpu.VMEM((A_CP, DN), jnp.float32),
        pltpu.VMEM((A_CP, DN), jnp.float32),
        pltpu.VMEM((A_CP, DN), jnp.float32),
        pltpu.VMEM((A_CP, DN), jnp.float32),
        pltpu.VMEM((A_CN, DN), jnp.float32),
        pltpu.VMEM((A_CN, DN), jnp.float32),
        pltpu.SemaphoreType.DMA,
        pltpu.SemaphoreType.DMA,
        pltpu.SemaphoreType.DMA,
        pltpu.SemaphoreType.DMA,
    ],
)
def _agg1_sc(nn_hbm, en_hbm, x_hbm, e_hbm, lp_hbm, mnx_hbm, mne_hbm,
             nnL, enL, xr0, xr1, er0, er1, ox_v, oe_v,
             semx0, semx1, seme0, seme1):
    del lp_hbm
    wid = _wid()
    scale = jnp.float32(1.0 / K)
    xr = (xr0, xr1)
    er = (er0, er1)
    semx = (semx0, semx1)
    seme = (seme0, seme1)

    base_pos = wid * (A_MAIN * A_CP)
    pltpu.sync_copy(nn_hbm.at[pl.ds(base_pos, A_MAIN * A_CP)],
                    nnL.at[pl.ds(0, A_MAIN * A_CP)])
    pltpu.sync_copy(en_hbm.at[pl.ds(base_pos, A_MAIN * A_CP)],
                    enL.at[pl.ds(0, A_MAIN * A_CP)])

    @pl.when(wid < A_TAIL)
    def _():
        tail_pos = (A_MAIN * NW + wid) * A_CP
        pltpu.sync_copy(nn_hbm.at[pl.ds(tail_pos, A_CP)],
                        nnL.at[pl.ds(A_MAIN * A_CP, A_CP)])
        pltpu.sync_copy(en_hbm.at[pl.ds(tail_pos, A_CP)],
                        enL.at[pl.ds(A_MAIN * A_CP, A_CP)])

    def valid(t):
        return (t < A_MAIN) | ((t == A_MAIN) & (wid < A_TAIL))

    def cid_of(t):
        return jnp.where(t == A_MAIN, A_MAIN * NW + wid, wid * A_MAIN + t)

    def issue(t, b):
        @pl.when(valid(t))
        def _():
            sl = pl.ds(t * A_CP, A_CP)
            pltpu.async_copy(x_hbm.at[nnL.at[sl]], xr[b], semx[b])
            pltpu.async_copy(e_hbm.at[enL.at[sl]], er[b], seme[b])

    def wait(t, b):
        @pl.when(valid(t))
        def _():
            sl = pl.ds(t * A_CP, A_CP)
            pltpu.make_async_copy(x_hbm.at[nnL.at[sl]], xr[b], semx[b]).wait()
            pltpu.make_async_copy(e_hbm.at[enL.at[sl]], er[b], seme[b]).wait()

    def compute(t, b):
        @pl.when(valid(t))
        def _():
            cid = cid_of(t)

            def node_body(i, carry2):
                base = i * K
                for cg in range(DN // 16):
                    sl = pl.ds(cg * 16, 16)
                    acc = xr[b][base, sl]
                    acce = er[b][base, sl]
                    for k in range(1, K):
                        acc = acc + xr[b][base + k, sl]
                        acce = acce + er[b][base + k, sl]
                    ox_v[i, sl] = acc * scale
                    oe_v[i, sl] = acce * scale
                return carry2

            lax.fori_loop(0, A_CN, node_body, 0)
            row_off = pl.multiple_of(cid * A_CN, 8)
            pltpu.sync_copy(ox_v, mnx_hbm.at[pl.ds(row_off, A_CN)])
            pltpu.sync_copy(oe_v, mne_hbm.at[pl.ds(row_off, A_CN)])

    issue(0, 0)

    def pair_body(tt, carry):
        for b in range(2):
            t = tt * 2 + b
            wait(t, b)
            issue(t + 1, 1 - b)
            compute(t, b)
        return carry

    lax.fori_loop(0, (A_MAIN + 1) // 2, pair_body, 0)


B_CN = 5
B_CP = B_CN * K
B_NCH = N // B_CN
B_MAIN = B_NCH // NW
B_TAIL = B_NCH - B_MAIN * NW
B_SLAB = (B_MAIN + 1) * B_CP


@functools.partial(
    pl.kernel,
    out_type=(
        jax.ShapeDtypeStruct((N * H,), jnp.float32),
        jax.ShapeDtypeStruct((N * DN,), jnp.float32),
    ),
    mesh=plsc.VectorSubcoreMesh(**_MESH),
    compiler_params=_SC_PARAMS,
    scratch_types=[
        pltpu.VMEM((B_SLAB,), jnp.int32),
        pltpu.VMEM((B_SLAB,), jnp.int32),
        pltpu.VMEM((B_SLAB,), jnp.int32),
        pltpu.VMEM((B_SLAB,), jnp.int32),
        pltpu.VMEM((B_SLAB,), jnp.int32),
        pltpu.VMEM((B_CP, H), jnp.float32),
        pltpu.VMEM((B_CP, H), jnp.float32),
        pltpu.VMEM((B_CP, H), jnp.float32),
        pltpu.VMEM((B_CP, H), jnp.float32),
        pltpu.VMEM((B_CP, H), jnp.float32),
        pltpu.VMEM((B_CP, H), jnp.float32),
        pltpu.VMEM((B_CP, DN), jnp.float32),
        pltpu.VMEM((B_CP, DN), jnp.float32),
        pltpu.VMEM((B_CN * H,), jnp.float32),
        pltpu.VMEM((B_CN * DN,), jnp.float32),
        pltpu.SemaphoreType.DMA,
        pltpu.SemaphoreType.DMA,
        pltpu.SemaphoreType.DMA,
        pltpu.SemaphoreType.DMA,
        pltpu.SemaphoreType.DMA,
        pltpu.SemaphoreType.DMA,
        pltpu.SemaphoreType.DMA,
        pltpu.SemaphoreType.DMA,
        pltpu.SemaphoreType.DMA,
        pltpu.SemaphoreType.DMA,
    ],
)
def _edge2_sc(nn_hbm, en_hbm, lp_hbm, a_hbm, b_hbm, c_hbm, x1_hbm,
              me2_hbm, mnx2_hbm,
              enL, nnL, gL, aiL, biL,
              ar0, ar1, br0, br1, cr0, cr1, xr0, xr1,
              ome_v, omx_v,
              semg, sembi, sa0, sa1, sb0, sb1, sc0, sc1, sx0, sx1):
    wid = _wid()
    scale = jnp.float32(1.0 / K)
    ar = (ar0, ar1)
    br = (br0, br1)
    cr = (cr0, cr1)
    xr = (xr0, xr1)
    sa = (sa0, sa1)
    sb = (sb0, sb1)
    sc = (sc0, sc1)
    sx = (sx0, sx1)

    main_pos = B_MAIN * B_CP
    base_pos = wid * main_pos
    pltpu.sync_copy(en_hbm.at[pl.ds(base_pos, main_pos)],
                    enL.at[pl.ds(0, main_pos)])
    pltpu.sync_copy(nn_hbm.at[pl.ds(base_pos, main_pos)],
                    nnL.at[pl.ds(0, main_pos)])

    @pl.when(wid < B_TAIL)
    def _():
        tail_pos = (B_MAIN * NW + wid) * B_CP
        pltpu.sync_copy(en_hbm.at[pl.ds(tail_pos, B_CP)],
                        enL.at[pl.ds(main_pos, B_CP)])
        pltpu.sync_copy(nn_hbm.at[pl.ds(tail_pos, B_CP)],
                        nnL.at[pl.ds(main_pos, B_CP)])

    def valid(t):
        return (t < B_MAIN) | ((t == B_MAIN) & (wid < B_TAIL))

    def cid_of(t):
        return jnp.where(t == B_MAIN, B_MAIN * NW + wid, wid * B_MAIN + t)

    n_trips = B_MAIN + 1

    def ga_issue(t, carry):
        @pl.when(valid(t))
        def _():
            sl = pl.ds(t * B_CP, B_CP)
            pltpu.async_copy(lp_hbm.at[enL.at[sl]], gL.at[sl], semg)
        return carry

    lax.fori_loop(0, n_trips, ga_issue, 0)

    def ga_drain(t, carry):
        @pl.when(valid(t))
        def _():
            sl = pl.ds(t * B_CP, B_CP)
            pltpu.make_async_copy(lp_hbm.at[enL.at[sl]], gL.at[sl],
                                  semg).wait()

            def sh_body(i, carry2):
                off = t * B_CP + i * 16
                aiL[pl.ds(off, 16)] = lax.shift_right_logical(
                    gL[pl.ds(off, 16)], 4)
                return carry2

            lax.fori_loop(0, B_CP // 16, sh_body, 0)
            pltpu.async_copy(nn_hbm.at[gL.at[sl]], biL.at[sl], sembi)
        return carry

    lax.fori_loop(0, n_trips, ga_drain, 0)

    def bi_drain(t, carry):
        @pl.when(valid(t))
        def _():
            sl = pl.ds(t * B_CP, B_CP)
            pltpu.make_async_copy(nn_hbm.at[gL.at[sl]], biL.at[sl],
                                  sembi).wait()
        return carry

    lax.fori_loop(0, n_trips, bi_drain, 0)

    def issue(t, b):
        @pl.when(valid(t))
        def _():
            sl = pl.ds(t * B_CP, B_CP)
            pltpu.async_copy(a_hbm.at[aiL.at[sl]], ar[b], sa[b])
            pltpu.async_copy(b_hbm.at[biL.at[sl]], br[b], sb[b])
            pltpu.async_copy(c_hbm.at[enL.at[sl]], cr[b], sc[b])
            pltpu.async_copy(x1_hbm.at[nnL.at[sl]], xr[b], sx[b])

    def wait(t, b):
        @pl.when(valid(t))
        def _():
            sl = pl.ds(t * B_CP, B_CP)
            pltpu.make_async_copy(a_hbm.at[aiL.at[sl]], ar[b], sa[b]).wait()
            pltpu.make_async_copy(b_hbm.at[biL.at[sl]], br[b], sb[b]).wait()
            pltpu.make_async_copy(c_hbm.at[enL.at[sl]], cr[b], sc[b]).wait()
            pltpu.make_async_copy(x1_hbm.at[nnL.at[sl]], xr[b], sx[b]).wait()

    def compute(t, b):
        @pl.when(valid(t))
        def _():
            cid = cid_of(t)

            def node_body(i, carry2):
                base = i * K
                for cg in range(H // 16):
                    sl = pl.ds(cg * 16, 16)
                    acce = jnp.maximum(
                        ar[b][base, sl] + br[b][base, sl] + cr[b][base, sl],
                        0.0)
                    accx = xr[b][base, sl]
                    for k in range(1, K):
                        acce = acce + jnp.maximum(
                            ar[b][base + k, sl] + br[b][base + k, sl]
                            + cr[b][base + k, sl], 0.0)
                        accx = accx + xr[b][base + k, sl]
                    ome_v[pl.ds(i * H + cg * 16, 16)] = acce * scale
                    omx_v[pl.ds(i * DN + cg * 16, 16)] = accx * scale
                return carry2

            lax.fori_loop(0, B_CN, node_body, 0)
            out_off = pl.multiple_of(cid * (B_CN * H), 128)
            pltpu.sync_copy(ome_v, me2_hbm.at[pl.ds(out_off, B_CN * H)])
            pltpu.sync_copy(omx_v, mnx2_hbm.at[pl.ds(out_off, B_CN * DN)])

    issue(0, 0)

    def pair_body(tt, carry):
        for b in range(2):
            t = tt * 2 + b
            wait(t, b)
            issue(t + 1, 1 - b)
            compute(t, b)
        return carry

    lax.fori_loop(0, (n_trips + 1) // 2, pair_body, 0)


_BN = 400
_BE = 2000


def _dot(a, b):
    return jax.lax.dot_general(a, b, (((1,), (0,)), ((), ())),
                               preferred_element_type=jnp.float32)


def _tc_ab_body(x_ref, wea_ref, web_ref, a_ref, b_ref):
    x = x_ref[...]
    a_ref[...] = _dot(x, wea_ref[...])
    b_ref[...] = _dot(x, web_ref[...])


def _tc_ab(x0, wea, web):
    row = lambda i: (i, 0)
    full = lambda i: (0, 0)
    return pl.pallas_call(
        _tc_ab_body,
        grid=(N // _BN,),
        in_specs=[
            pl.BlockSpec((_BN, DN), row),
            pl.BlockSpec((DN, H), full),
            pl.BlockSpec((DN, H), full),
        ],
        out_specs=[
            pl.BlockSpec((_BN, H), row),
            pl.BlockSpec((_BN, H), row),
        ],
        out_shape=[
            jax.ShapeDtypeStruct((N, H), jnp.float32),
            jax.ShapeDtypeStruct((N, H), jnp.float32),
        ],
    )(x0, wea, web)


def _tc_node1_body(x_ref, mnx_ref, mnd_ref, wna_ref, wnb_ref, bn_ref,
                   w2a_ref, x1_ref, p_ref):
    acc = _dot(x_ref[...], wna_ref[...])
    acc = acc + _dot(mnx_ref[...], wnb_ref[...])
    x1 = jnp.maximum(acc + mnd_ref[...] + bn_ref[...], 0.0)
    x1_ref[...] = x1
    p_ref[...] = _dot(x1, w2a_ref[...])


def _tc_node1(x0, mnx1, mnd1, wna, wnb, bn1, w2a):
    row = lambda i: (i, 0)
    full = lambda i: (0, 0)
    return pl.pallas_call(
        _tc_node1_body,
        grid=(N // _BN,),
        in_specs=[
            pl.BlockSpec((_BN, DN), row),
            pl.BlockSpec((_BN, DN), row),
            pl.BlockSpec((_BN, H), row),
            pl.BlockSpec((DN, H), full),
            pl.BlockSpec((DN, H), full),
            pl.BlockSpec((1, H), full),
            pl.BlockSpec((H, H), full),
        ],
        out_specs=[
            pl.BlockSpec((_BN, H), row),
            pl.BlockSpec((_BN, H), row),
        ],
        out_shape=[
            jax.ShapeDtypeStruct((N, H), jnp.float32),
            jax.ShapeDtypeStruct((N, H), jnp.float32),
        ],
    )(x0, mnx1, mnd1, wna, wnb, bn1, w2a)


def _tc_edge1_body(e_ref, wec_ref, be_ref, wnc_ref, c_ref, d_ref):
    e = e_ref[...]
    c_ref[...] = _dot(e, wec_ref[...]) + be_ref[...]
    d_ref[...] = _dot(e, wnc_ref[...])


def _tc_edge1(e0, wec, be1, wnc):
    return pl.pallas_call(
        _tc_edge1_body,
        grid=(E // _BE,),
        in_specs=[
            pl.BlockSpec((_BE, DE), lambda i: (i, 0)),
            pl.BlockSpec((DE, H), lambda i: (0, 0)),
            pl.BlockSpec((1, H), lambda i: (0, 0)),
            pl.BlockSpec((DE, H), lambda i: (0, 0)),
        ],
        out_specs=[
            pl.BlockSpec((_BE, H), lambda i: (i, 0)),
            pl.BlockSpec((_BE, H), lambda i: (i, 0)),
        ],
        out_shape=[
            jax.ShapeDtypeStruct((E, H), jnp.float32),
            jax.ShapeDtypeStruct((E, H), jnp.float32),
        ],
    )(e0, wec, be1, wnc)


def _tc_node2_body(p_ref, mnx_ref, me_ref, wb_ref, wc_ref, bn_ref,
                   wf_ref, bf_ref, out_ref):
    acc = p_ref[...]
    acc = acc + _dot(mnx_ref[...], wb_ref[...])
    acc = acc + _dot(me_ref[...], wc_ref[...])
    x2 = jnp.maximum(acc + bn_ref[...], 0.0)
    out_ref[...] = _dot(x2, wf_ref[...]) + bf_ref[...]


def _tc_node2(p1, mnx2, me2, wb, wc, bn2, wf, bf):
    row = lambda i: (i, 0)
    full = lambda i: (0, 0)
    return pl.pallas_call(
        _tc_node2_body,
        grid=(N // _BN,),
        in_specs=[
            pl.BlockSpec((_BN, H), row),
            pl.BlockSpec((_BN, DN), row),
            pl.BlockSpec((_BN, H), row),
            pl.BlockSpec((DN, H), full),
            pl.BlockSpec((H, H), full),
            pl.BlockSpec((1, H), full),
            pl.BlockSpec((H, H), full),
            pl.BlockSpec((1, H), full),
        ],
        out_specs=pl.BlockSpec((_BN, H), row),
        out_shape=jax.ShapeDtypeStruct((N, H), jnp.float32),
    )(p1, mnx2, me2, wb, wc, bn2, wf, bf)


def kernel(node_neighbors, edge_neighbors, node_feats, edge_feats,
           Wn1, bn1, We1, be1, Wn2, bn2, We2, be2, Wf, bf):
    nn_f = node_neighbors.reshape(-1).astype(jnp.int32)
    en_f = edge_neighbors.reshape(-1).astype(jnp.int32)
    x0 = node_feats
    e0 = edge_feats

    lastpos = _lastpos_sc(en_f)
    C1, D1 = _tc_edge1(e0, We1[2 * DN:], be1.reshape(1, H), Wn1[2 * DN:])
    A1, B1 = _tc_ab(x0, We1[:DN], We1[DN:2 * DN])

    mnx1, mnd1 = _agg1_sc(nn_f, en_f, x0, D1, lastpos)

    x1, P1 = _tc_node1(x0, mnx1, mnd1, Wn1[:DN], Wn1[DN:2 * DN],
                       bn1.reshape(1, H), Wn2[:H])

    me2_f, mnx2_f = _edge2_sc(nn_f, en_f, lastpos, A1, B1, C1, x1)
    me2 = me2_f.reshape(N, H)
    mnx2 = mnx2_f.reshape(N, DN)

    out = _tc_node2(
        P1, mnx2, me2,
        Wn2[H:2 * H], Wn2[2 * H:],
        bn2.reshape(1, H), Wf, bf.reshape(1, H))
    return out


# speedup vs baseline: 1.2342x; 1.0021x over previous
import functools

import jax
import jax.numpy as jnp
from jax import lax
from jax.experimental import pallas as pl
from jax.experimental.pallas import tpu as pltpu
from jax.experimental.pallas import tpu_sc as plsc

N = 10000
K = 16
E = 160000
DN = 128
DE = 16
H = 128
NK = N * K

NC = 2
NS = 16
NW = NC * NS

_MESH = dict(core_axis_name="c", subcore_axis_name="s")
_SC_PARAMS = pltpu.CompilerParams(needs_layout_passes=False)

IDS_PER_W = E // NW
LP_CHUNK = 16000
LP_NCH = NK // LP_CHUNK


def _wid():
    return lax.axis_index("s") * NC + lax.axis_index("c")


@functools.partial(
    pl.kernel,
    out_type=jax.ShapeDtypeStruct((E,), jnp.int32),
    mesh=plsc.VectorSubcoreMesh(**_MESH),
    compiler_params=_SC_PARAMS,
    scratch_types=[
        pltpu.VMEM((IDS_PER_W,), jnp.int32),
        pltpu.VMEM((LP_CHUNK,), jnp.int32),
        pltpu.VMEM((LP_CHUNK,), jnp.int32),
        pltpu.SemaphoreType.DMA,
        pltpu.SemaphoreType.DMA,
    ],
)
def _lastpos_sc(en_hbm, out_hbm, lp_v, ids0, ids1, sem0, sem1):
    wid = _wid()
    lo = wid * IDS_PER_W
    iota = lax.iota(jnp.int32, 16)
    ids = (ids0, ids1)
    sem = (sem0, sem1)

    def issue(c, b):
        @pl.when(c < LP_NCH)
        def _():
            pltpu.async_copy(en_hbm.at[pl.ds(c * LP_CHUNK, LP_CHUNK)],
                             ids[b], sem[b])

    def wait(c, b):
        pltpu.make_async_copy(en_hbm.at[pl.ds(c * LP_CHUNK, LP_CHUNK)],
                              ids[b], sem[b]).wait()

    def compute(c, b):
        ids_v = ids[b]

        def vec_body(i, carry2):
            for u in range(8):
                off = i * 128 + u * 16
                v = ids_v[pl.ds(off, 16)]
                local = v - lo
                m = local.astype(jnp.uint32) < jnp.uint32(IDS_PER_W)
                _, last = plsc.scan_count(v, mask=m)
                pos = (c * LP_CHUNK + off) + iota
                plsc.store_scatter(lp_v, [local], pos, mask=last)
            return carry2

        lax.fori_loop(0, LP_CHUNK // 128, vec_body, 0)

    issue(0, 0)

    def pair_body(cc, carry):
        for b in range(2):
            c = cc * 2 + b
            wait(c, b)
            issue(c + 1, 1 - b)
            compute(c, b)
        return carry

    lax.fori_loop(0, LP_NCH // 2, pair_body, 0)
    pltpu.sync_copy(lp_v, out_hbm.at[pl.ds(lo, IDS_PER_W)])


A_CN = 8
A_CP = A_CN * K
A_NCH = N // A_CN
A_MAIN = A_NCH // NW
A_TAIL = A_NCH - A_MAIN * NW
A_SLAB = (A_MAIN + 1) * A_CP


@functools.partial(
    pl.kernel,
    out_type=(
        jax.ShapeDtypeStruct((N, DN), jnp.float32),
        jax.ShapeDtypeStruct((N, DN), jnp.float32),
    ),
    mesh=plsc.VectorSubcoreMesh(**_MESH),
    compiler_params=_SC_PARAMS,
    scratch_types=[
        pltpu.VMEM((A_SLAB,), jnp.int32),
        pltpu.VMEM((A_SLAB,), jnp.int32),
        pltpu.VMEM((A_CP, DN), jnp.float32),
        pltpu.VMEM((A_CP, DN), jnp.float32),
        pltpu.VMEM((A_CP, DN), jnp.float32),
        pltpu.VMEM((A_CP, DN), jnp.float32),
        pltpu.VMEM((A_CN, DN), jnp.float32),
        pltpu.VMEM((A_CN, DN), jnp.float32),
        pltpu.SemaphoreType.DMA,
        pltpu.SemaphoreType.DMA,
        pltpu.SemaphoreType.DMA,
        pltpu.SemaphoreType.DMA,
    ],
)
def _agg1_sc(nn_hbm, en_hbm, x_hbm, e_hbm, lp_hbm, mnx_hbm, mne_hbm,
             nnL, enL, xr0, xr1, er0, er1, ox_v, oe_v,
             semx0, semx1, seme0, seme1):
    del lp_hbm
    wid = _wid()
    scale = jnp.float32(1.0 / K)
    xr = (xr0, xr1)
    er = (er0, er1)
    semx = (semx0, semx1)
    seme = (seme0, seme1)

    base_pos = wid * (A_MAIN * A_CP)
    pltpu.sync_copy(nn_hbm.at[pl.ds(base_pos, A_MAIN * A_CP)],
                    nnL.at[pl.ds(0, A_MAIN * A_CP)])
    pltpu.sync_copy(en_hbm.at[pl.ds(base_pos, A_MAIN * A_CP)],
                    enL.at[pl.ds(0, A_MAIN * A_CP)])

    @pl.when(wid < A_TAIL)
    def _():
        tail_pos = (A_MAIN * NW + wid) * A_CP
        pltpu.sync_copy(nn_hbm.at[pl.ds(tail_pos, A_CP)],
                        nnL.at[pl.ds(A_MAIN * A_CP, A_CP)])
        pltpu.sync_copy(en_hbm.at[pl.ds(tail_pos, A_CP)],
                        enL.at[pl.ds(A_MAIN * A_CP, A_CP)])

    def valid(t):
        return (t < A_MAIN) | ((t == A_MAIN) & (wid < A_TAIL))

    def cid_of(t):
        return jnp.where(t == A_MAIN, A_MAIN * NW + wid, wid * A_MAIN + t)

    def issue(t, b):
        @pl.when(valid(t))
        def _():
            sl = pl.ds(t * A_CP, A_CP)
            pltpu.async_copy(x_hbm.at[nnL.at[sl]], xr[b], semx[b])
            pltpu.async_copy(e_hbm.at[enL.at[sl]], er[b], seme[b])

    def wait(t, b):
        @pl.when(valid(t))
        def _():
            sl = pl.ds(t * A_CP, A_CP)
            pltpu.make_async_copy(x_hbm.at[nnL.at[sl]], xr[b], semx[b]).wait()
            pltpu.make_async_copy(e_hbm.at[enL.at[sl]], er[b], seme[b]).wait()

    def compute(t, b):
        @pl.when(valid(t))
        def _():
            cid = cid_of(t)

            def node_body(i, carry2):
                base = i * K
                for cg in range(DN // 16):
                    sl = pl.ds(cg * 16, 16)
                    acc = xr[b][base, sl]
                    acce = er[b][base, sl]
                    for k in range(1, K):
                        acc = acc + xr[b][base + k, sl]
                        acce = acce + er[b][base + k, sl]
                    ox_v[i, sl] = acc * scale
                    oe_v[i, sl] = acce * scale
                return carry2

            lax.fori_loop(0, A_CN, node_body, 0)
            row_off = pl.multiple_of(cid * A_CN, 8)
            pltpu.sync_copy(ox_v, mnx_hbm.at[pl.ds(row_off, A_CN)])
            pltpu.sync_copy(oe_v, mne_hbm.at[pl.ds(row_off, A_CN)])

    issue(0, 0)

    def pair_body(tt, carry):
        for b in range(2):
            t = tt * 2 + b
            wait(t, b)
            issue(t + 1, 1 - b)
            compute(t, b)
        return carry

    lax.fori_loop(0, (A_MAIN + 1) // 2, pair_body, 0)


B_CN = 5
B_CP = B_CN * K
B_NCH = N // B_CN
B_MAIN = B_NCH // NW
B_TAIL = B_NCH - B_MAIN * NW
B_SLAB = (B_MAIN + 1) * B_CP


@functools.partial(
    pl.kernel,
    out_type=(
        jax.ShapeDtypeStruct((N * H,), jnp.float32),
        jax.ShapeDtypeStruct((N * DN,), jnp.float32),
    ),
    mesh=plsc.VectorSubcoreMesh(**_MESH),
    compiler_params=_SC_PARAMS,
    scratch_types=[
        pltpu.VMEM((B_SLAB,), jnp.int32),
        pltpu.VMEM((B_SLAB,), jnp.int32),
        pltpu.VMEM((B_SLAB,), jnp.int32),
        pltpu.VMEM((B_SLAB,), jnp.int32),
        pltpu.VMEM((B_SLAB,), jnp.int32),
        pltpu.VMEM((B_CP, H), jnp.float32),
        pltpu.VMEM((B_CP, H), jnp.float32),
        pltpu.VMEM((B_CP, H), jnp.float32),
        pltpu.VMEM((B_CP, H), jnp.float32),
        pltpu.VMEM((B_CP, H), jnp.float32),
        pltpu.VMEM((B_CP, H), jnp.float32),
        pltpu.VMEM((B_CP, DN), jnp.float32),
        pltpu.VMEM((B_CP, DN), jnp.float32),
        pltpu.VMEM((B_CN * H,), jnp.float32),
        pltpu.VMEM((B_CN * DN,), jnp.float32),
        pltpu.SemaphoreType.DMA,
        pltpu.SemaphoreType.DMA,
        pltpu.SemaphoreType.DMA,
        pltpu.SemaphoreType.DMA,
        pltpu.SemaphoreType.DMA,
        pltpu.SemaphoreType.DMA,
        pltpu.SemaphoreType.DMA,
        pltpu.SemaphoreType.DMA,
        pltpu.SemaphoreType.DMA,
        pltpu.SemaphoreType.DMA,
    ],
)
def _edge2_sc(nn_hbm, en_hbm, lp_hbm, a_hbm, b_hbm, c_hbm, x1_hbm,
              me2_hbm, mnx2_hbm,
              enL, nnL, gL, aiL, biL,
              ar0, ar1, br0, br1, cr0, cr1, xr0, xr1,
              ome_v, omx_v,
              semg, sembi, sa0, sa1, sb0, sb1, sc0, sc1, sx0, sx1):
    wid = _wid()
    scale = jnp.float32(1.0 / K)
    ar = (ar0, ar1)
    br = (br0, br1)
    cr = (cr0, cr1)
    xr = (xr0, xr1)
    sa = (sa0, sa1)
    sb = (sb0, sb1)
    sc = (sc0, sc1)
    sx = (sx0, sx1)

    main_pos = B_MAIN * B_CP
    base_pos = wid * main_pos
    pltpu.sync_copy(en_hbm.at[pl.ds(base_pos, main_pos)],
                    enL.at[pl.ds(0, main_pos)])
    pltpu.sync_copy(nn_hbm.at[pl.ds(base_pos, main_pos)],
                    nnL.at[pl.ds(0, main_pos)])

    @pl.when(wid < B_TAIL)
    def _():
        tail_pos = (B_MAIN * NW + wid) * B_CP
        pltpu.sync_copy(en_hbm.at[pl.ds(tail_pos, B_CP)],
                        enL.at[pl.ds(main_pos, B_CP)])
        pltpu.sync_copy(nn_hbm.at[pl.ds(tail_pos, B_CP)],
                        nnL.at[pl.ds(main_pos, B_CP)])

    def valid(t):
        return (t < B_MAIN) | ((t == B_MAIN) & (wid < B_TAIL))

    def cid_of(t):
        return jnp.where(t == B_MAIN, B_MAIN * NW + wid, wid * B_MAIN + t)

    n_trips = B_MAIN + 1

    def ga_issue(t, carry):
        @pl.when(valid(t))
        def _():
            sl = pl.ds(t * B_CP, B_CP)
            pltpu.async_copy(lp_hbm.at[enL.at[sl]], gL.at[sl], semg)
        return carry

    lax.fori_loop(0, n_trips, ga_issue, 0)

    def ga_drain(t, carry):
        @pl.when(valid(t))
        def _():
            sl = pl.ds(t * B_CP, B_CP)
            pltpu.make_async_copy(lp_hbm.at[enL.at[sl]], gL.at[sl],
                                  semg).wait()

            def sh_body(i, carry2):
                off = t * B_CP + i * 16
                aiL[pl.ds(off, 16)] = lax.shift_right_logical(
                    gL[pl.ds(off, 16)], 4)
                return carry2

            lax.fori_loop(0, B_CP // 16, sh_body, 0)
            pltpu.async_copy(nn_hbm.at[gL.at[sl]], biL.at[sl], sembi)
        return carry

    lax.fori_loop(0, n_trips, ga_drain, 0)

    def bi_drain(t, carry):
        @pl.when(valid(t))
        def _():
            sl = pl.ds(t * B_CP, B_CP)
            pltpu.make_async_copy(nn_hbm.at[gL.at[sl]], biL.at[sl],
                                  sembi).wait()
        return carry

    lax.fori_loop(0, n_trips, bi_drain, 0)

    def issue(t, b):
        @pl.when(valid(t))
        def _():
            sl = pl.ds(t * B_CP, B_CP)
            pltpu.async_copy(a_hbm.at[aiL.at[sl]], ar[b], sa[b])
            pltpu.async_copy(b_hbm.at[biL.at[sl]], br[b], sb[b])
            pltpu.async_copy(c_hbm.at[enL.at[sl]], cr[b], sc[b])
            pltpu.async_copy(x1_hbm.at[nnL.at[sl]], xr[b], sx[b])

    def wait(t, b):
        @pl.when(valid(t))
        def _():
            sl = pl.ds(t * B_CP, B_CP)
            pltpu.make_async_copy(a_hbm.at[aiL.at[sl]], ar[b], sa[b]).wait()
            pltpu.make_async_copy(b_hbm.at[biL.at[sl]], br[b], sb[b]).wait()
            pltpu.make_async_copy(c_hbm.at[enL.at[sl]], cr[b], sc[b]).wait()
            pltpu.make_async_copy(x1_hbm.at[nnL.at[sl]], xr[b], sx[b]).wait()

    def compute(t, b):
        @pl.when(valid(t))
        def _():
            cid = cid_of(t)

            def node_body(i, carry2):
                base = i * K
                for cg in range(H // 16):
                    sl = pl.ds(cg * 16, 16)
                    acce = jnp.maximum(
                        ar[b][base, sl] + br[b][base, sl] + cr[b][base, sl],
                        0.0)
                    accx = xr[b][base, sl]
                    for k in range(1, K):
                        acce = acce + jnp.maximum(
                            ar[b][base + k, sl] + br[b][base + k, sl]
                            + cr[b][base + k, sl], 0.0)
                        accx = accx + xr[b][base + k, sl]
                    ome_v[pl.ds(i * H + cg * 16, 16)] = acce * scale
                    omx_v[pl.ds(i * DN + cg * 16, 16)] = accx * scale
                return carry2

            lax.fori_loop(0, B_CN, node_body, 0)
            out_off = pl.multiple_of(cid * (B_CN * H), 128)
            pltpu.sync_copy(ome_v, me2_hbm.at[pl.ds(out_off, B_CN * H)])
            pltpu.sync_copy(omx_v, mnx2_hbm.at[pl.ds(out_off, B_CN * DN)])

    issue(0, 0)

    def pair_body(tt, carry):
        for b in range(2):
            t = tt * 2 + b
            wait(t, b)
            issue(t + 1, 1 - b)
            compute(t, b)
        return carry

    lax.fori_loop(0, (n_trips + 1) // 2, pair_body, 0)


_BN = 400
_BE = 2000


def _dot(a, b):
    return jax.lax.dot_general(a, b, (((1,), (0,)), ((), ())),
                               preferred_element_type=jnp.float32)


def _tc_ab_body(x_ref, wea_ref, web_ref, a_ref, b_ref):
    x = x_ref[...]
    a_ref[...] = _dot(x, wea_ref[...])
    b_ref[...] = _dot(x, web_ref[...])


def _tc_ab(x0, wea, web):
    row = lambda i: (i, 0)
    full = lambda i: (0, 0)
    return pl.pallas_call(
        _tc_ab_body,
        grid=(N // _BN,),
        in_specs=[
            pl.BlockSpec((_BN, DN), row),
            pl.BlockSpec((DN, H), full),
            pl.BlockSpec((DN, H), full),
        ],
        out_specs=[
            pl.BlockSpec((_BN, H), row),
            pl.BlockSpec((_BN, H), row),
        ],
        out_shape=[
            jax.ShapeDtypeStruct((N, H), jnp.float32),
            jax.ShapeDtypeStruct((N, H), jnp.float32),
        ],
    )(x0, wea, web)


def _tc_node1_body(x_ref, mnx_ref, mnd_ref, wna_ref, wnb_ref, bn_ref,
                   x1_ref):
    acc = _dot(x_ref[...], wna_ref[...])
    acc = acc + _dot(mnx_ref[...], wnb_ref[...])
    x1_ref[...] = jnp.maximum(acc + mnd_ref[...] + bn_ref[...], 0.0)


def _tc_node1(x0, mnx1, mnd1, wna, wnb, bn1):
    row = lambda i: (i, 0)
    full = lambda i: (0, 0)
    return pl.pallas_call(
        _tc_node1_body,
        grid=(N // _BN,),
        in_specs=[
            pl.BlockSpec((_BN, DN), row),
            pl.BlockSpec((_BN, DN), row),
            pl.BlockSpec((_BN, H), row),
            pl.BlockSpec((DN, H), full),
            pl.BlockSpec((DN, H), full),
            pl.BlockSpec((1, H), full),
        ],
        out_specs=pl.BlockSpec((_BN, H), row),
        out_shape=jax.ShapeDtypeStruct((N, H), jnp.float32),
    )(x0, mnx1, mnd1, wna, wnb, bn1)


def _tc_p1_body(x1_ref, w2a_ref, p_ref):
    p_ref[...] = _dot(x1_ref[...], w2a_ref[...])


def _tc_p1(x1, w2a):
    return pl.pallas_call(
        _tc_p1_body,
        grid=(N // _BN,),
        in_specs=[
            pl.BlockSpec((_BN, H), lambda i: (i, 0)),
            pl.BlockSpec((H, H), lambda i: (0, 0)),
        ],
        out_specs=pl.BlockSpec((_BN, H), lambda i: (i, 0)),
        out_shape=jax.ShapeDtypeStruct((N, H), jnp.float32),
    )(x1, w2a)


def _tc_edge1_body(e_ref, wec_ref, be_ref, wnc_ref, c_ref, d_ref):
    e = e_ref[...]
    c_ref[...] = _dot(e, wec_ref[...]) + be_ref[...]
    d_ref[...] = _dot(e, wnc_ref[...])


def _tc_edge1(e0, wec, be1, wnc):
    return pl.pallas_call(
        _tc_edge1_body,
        grid=(E // _BE,),
        in_specs=[
            pl.BlockSpec((_BE, DE), lambda i: (i, 0)),
            pl.BlockSpec((DE, H), lambda i: (0, 0)),
            pl.BlockSpec((1, H), lambda i: (0, 0)),
            pl.BlockSpec((DE, H), lambda i: (0, 0)),
        ],
        out_specs=[
            pl.BlockSpec((_BE, H), lambda i: (i, 0)),
            pl.BlockSpec((_BE, H), lambda i: (i, 0)),
        ],
        out_shape=[
            jax.ShapeDtypeStruct((E, H), jnp.float32),
            jax.ShapeDtypeStruct((E, H), jnp.float32),
        ],
    )(e0, wec, be1, wnc)


def _tc_node2_body(p_ref, mnx_ref, me_ref, wb_ref, wc_ref, bn_ref,
                   wf_ref, bf_ref, out_ref):
    acc = p_ref[...]
    acc = acc + _dot(mnx_ref[...], wb_ref[...])
    acc = acc + _dot(me_ref[...], wc_ref[...])
    x2 = jnp.maximum(acc + bn_ref[...], 0.0)
    out_ref[...] = _dot(x2, wf_ref[...]) + bf_ref[...]


def _tc_node2(p1, mnx2, me2, wb, wc, bn2, wf, bf):
    row = lambda i: (i, 0)
    full = lambda i: (0, 0)
    return pl.pallas_call(
        _tc_node2_body,
        grid=(N // _BN,),
        in_specs=[
            pl.BlockSpec((_BN, H), row),
            pl.BlockSpec((_BN, DN), row),
            pl.BlockSpec((_BN, H), row),
            pl.BlockSpec((DN, H), full),
            pl.BlockSpec((H, H), full),
            pl.BlockSpec((1, H), full),
            pl.BlockSpec((H, H), full),
            pl.BlockSpec((1, H), full),
        ],
        out_specs=pl.BlockSpec((_BN, H), row),
        out_shape=jax.ShapeDtypeStruct((N, H), jnp.float32),
    )(p1, mnx2, me2, wb, wc, bn2, wf, bf)


def kernel(node_neighbors, edge_neighbors, node_feats, edge_feats,
           Wn1, bn1, We1, be1, Wn2, bn2, We2, be2, Wf, bf):
    nn_f = node_neighbors.reshape(-1).astype(jnp.int32)
    en_f = edge_neighbors.reshape(-1).astype(jnp.int32)
    x0 = node_feats
    e0 = edge_feats

    lastpos = _lastpos_sc(en_f)
    C1, D1 = _tc_edge1(e0, We1[2 * DN:], be1.reshape(1, H), Wn1[2 * DN:])
    A1, B1 = _tc_ab(x0, We1[:DN], We1[DN:2 * DN])

    mnx1, mnd1 = _agg1_sc(nn_f, en_f, x0, D1, lastpos)

    x1 = _tc_node1(x0, mnx1, mnd1, Wn1[:DN], Wn1[DN:2 * DN],
                   bn1.reshape(1, H))

    me2_f, mnx2_f = _edge2_sc(nn_f, en_f, lastpos, A1, B1, C1, x1)
    P1 = _tc_p1(x1, Wn2[:H])
    me2 = me2_f.reshape(N, H)
    mnx2 = mnx2_f.reshape(N, DN)

    out = _tc_node2(
        P1, mnx2, me2,
        Wn2[H:2 * H], Wn2[2 * H:],
        bn2.reshape(1, H), Wf, bf.reshape(1, H))
    return out
